# Initial kernel scaffold; baseline (speedup 1.0000x reference)
import jax
import jax.numpy as jnp
from jax.experimental import pallas as pl


def kernel(x, adj, W1, b1, W2, b2, W3, b3, bn_gamma, bn_beta, Wout, bout):
    raise NotImplementedError("write your pallas kernel here")


# speedup vs baseline: 1.2031x; 1.2031x over previous
import functools

import jax
import jax.numpy as jnp
from jax.experimental import pallas as pl

N = 2048
D_IN = 128
D_H = 128
D_OUT = 64
BN_EPS = 1e-5


def _gcn_fused_kernel(x_ref, adj_ref, W1_ref, b1_ref, W2_ref, b2_ref,
                      W3_ref, b3_ref, g_ref, be_ref, Wout_ref, bout_ref,
                      out_ref):
    adj = adj_ref[...]
    h = x_ref[...]
    layer_refs = ((W1_ref, b1_ref), (W2_ref, b2_ref), (W3_ref, b3_ref))
    for i, (W_ref, b_ref) in enumerate(layer_refs):
        support = jnp.dot(h, W_ref[...], preferred_element_type=jnp.float32)
        out = jnp.dot(adj, support, preferred_element_type=jnp.float32)
        out = jnp.maximum(out + b_ref[...], 0.0)
        mean = jnp.mean(out, axis=0, keepdims=True)
        var = jnp.mean((out - mean) ** 2, axis=0, keepdims=True)
        out = (out - mean) / jnp.sqrt(var + BN_EPS)
        h = out * g_ref[i, :][None, :] + be_ref[i, :][None, :]
    out_ref[...] = (jnp.dot(h, Wout_ref[...], preferred_element_type=jnp.float32)
                    + bout_ref[...])


@functools.partial(jax.jit, static_argnames=())
def kernel(x, adj, W1, b1, W2, b2, W3, b3, bn_gamma, bn_beta, Wout, bout):
    out = pl.pallas_call(
        _gcn_fused_kernel,
        out_shape=jax.ShapeDtypeStruct((N, D_OUT), jnp.float32),
    )(x, adj, W1, b1.reshape(1, D_H), W2, b2.reshape(1, D_H),
      W3, b3.reshape(1, D_H), bn_gamma, bn_beta, Wout,
      bout.reshape(1, D_OUT))
    return out
